# baseline (speedup 1.0000x reference)
import functools

import jax
import jax.numpy as jnp
from jax import lax
from jax.experimental import pallas as pl
from jax.experimental.pallas import tpu as pltpu
from jax.experimental.pallas import tpu_sc as plsc

N_NODES = 100000
N_DOWN = 25000
SPIRAL_LEN = 9
IN_C = 64
OUT_C = 64
NNZ = 100000

NW = 32
NPW = 3128
N_PAD = NW * NPW
CB = 136
RG = CB * SPIRAL_LEN

RPW = 784
OUT_PAD = NW * RPW
CE = 128
NNZ_PAD = NNZ + 2 * CE

_mesh = plsc.VectorSubcoreMesh(core_axis_name="c", subcore_axis_name="s")


def _sget(ref, i):
    return ref[pl.ds(i, 16)][0]


def _mm_body(x_ref, w_ref, o_ref):
    o_ref[...] = jnp.dot(x_ref[...], w_ref[0],
                         preferred_element_type=jnp.float32)


def _project(x2d, w3):
    bn = 1000
    return pl.pallas_call(
        _mm_body,
        grid=(N_NODES // bn, SPIRAL_LEN),
        in_specs=[
            pl.BlockSpec((bn, IN_C), lambda i, l: (i, 0)),
            pl.BlockSpec((1, IN_C, OUT_C), lambda i, l: (l, 0, 0)),
        ],
        out_specs=pl.BlockSpec((bn, OUT_C),
                               lambda i, l: (l * (N_NODES // bn) + i, 0)),
        out_shape=jax.ShapeDtypeStruct((N_NODES * SPIRAL_LEN, OUT_C),
                                       jnp.float32),
    )(x2d, w3)


@functools.partial(
    pl.kernel,
    mesh=_mesh,
    compiler_params=pltpu.CompilerParams(use_tc_tiling_on_sc=False),
    out_type=jax.ShapeDtypeStruct((N_PAD, OUT_C), jnp.float32),
    scratch_types=[
        pltpu.VMEM((RG,), jnp.int32),
        pltpu.VMEM((RG, OUT_C), jnp.float32),
        pltpu.VMEM((CB, OUT_C), jnp.float32),
        pltpu.VMEM((OUT_C,), jnp.float32),
        pltpu.SemaphoreType.DMA,
    ],
)
def _spiral(table_hbm, idx_hbm, b_hbm, h_hbm, idx_v, g_v, h_v, b_v, sem):
    wid = lax.axis_index("s") * 2 + lax.axis_index("c")
    base_node = wid * NPW
    pltpu.sync_copy(b_hbm, b_v)

    def chunk_body(ci, carry):
        nbase = base_node + ci * CB
        pltpu.sync_copy(idx_hbm.at[pl.ds(nbase * SPIRAL_LEN, RG)], idx_v)
        cps = []
        for s in range(9):
            cps.append(pltpu.async_copy(
                table_hbm.at[idx_v.at[pl.ds(s * 128, 128)]],
                g_v.at[pl.ds(s * 128, 128)], sem))
        cps.append(pltpu.async_copy(
            table_hbm.at[idx_v.at[pl.ds(1152, 72)]],
            g_v.at[pl.ds(1152, 72)], sem))
        for cp in cps:
            cp.wait()

        def node_body(c, carry2):
            r0 = c * SPIRAL_LEN
            for j in range(4):
                sl = pl.ds(j * 16, 16)
                v = g_v[r0, sl]
                for l in range(1, SPIRAL_LEN):
                    v = v + g_v[r0 + l, sl]
                v = v + b_v[sl]
                v = jnp.where(v > 0.0, v, jnp.exp(v) - 1.0)
                h_v[c, sl] = v
            return carry2

        lax.fori_loop(0, CB, node_body, 0)
        pltpu.sync_copy(h_v, h_hbm.at[pl.ds(nbase, CB)])
        return carry

    lax.fori_loop(0, NPW // CB, chunk_body, 0)


@functools.partial(
    pl.kernel,
    mesh=_mesh,
    compiler_params=pltpu.CompilerParams(use_tc_tiling_on_sc=False),
    out_type=jax.ShapeDtypeStruct((OUT_PAD, OUT_C), jnp.float32),
    scratch_types=[
        pltpu.VMEM((NW + 1 + 16,), jnp.int32),
        pltpu.VMEM((CE + 16,), jnp.int32),
        pltpu.VMEM((CE + 16,), jnp.float32),
        pltpu.VMEM((CE + 16,), jnp.int32),
        pltpu.VMEM((CE, OUT_C), jnp.float32),
        pltpu.VMEM((RPW, OUT_C), jnp.float32),
        pltpu.SemaphoreType.DMA,
    ],
)
def _pool(h_hbm, col_hbm, val_hbm, row_hbm, bnd_hbm, out_hbm,
          bnd_v, col_v, val_v, row_v, g_v, acc_v, sem):
    wid = lax.axis_index("s") * 2 + lax.axis_index("c")
    rbase = wid * RPW
    pltpu.sync_copy(bnd_hbm, bnd_v.at[pl.ds(0, 40)])
    k0 = _sget(bnd_v, wid)
    k1 = _sget(bnd_v, wid + 1)
    k0a = jnp.bitwise_and(k0, -8)
    nchunks = lax.shift_right_logical(k1 - k0a + (CE - 1), 7)

    zero16 = jnp.zeros((16,), jnp.float32)

    def zero_body(r, carry):
        for j in range(4):
            acc_v[r, pl.ds(j * 16, 16)] = zero16
        return carry

    lax.fori_loop(0, RPW, zero_body, 0)

    def chunk_body(ci, carry):
        kc = pl.multiple_of(k0a + ci * CE, 8)
        pltpu.sync_copy(col_hbm.at[pl.ds(kc, CE)], col_v.at[pl.ds(0, CE)])
        pltpu.sync_copy(val_hbm.at[pl.ds(kc, CE)], val_v.at[pl.ds(0, CE)])
        pltpu.sync_copy(row_hbm.at[pl.ds(kc, CE)], row_v.at[pl.ds(0, CE)])
        pltpu.async_copy(h_hbm.at[col_v.at[pl.ds(0, CE)]], g_v, sem).wait()

        def e_body(e, carry2):
            kg = kc + e
            ok = jnp.logical_and(kg >= k0, kg < k1)
            vm = jnp.where(ok, _sget(val_v, e), 0.0)
            rl = jnp.clip(_sget(row_v, e) - rbase, 0, RPW - 1)
            for j in range(4):
                sl = pl.ds(j * 16, 16)
                acc_v[rl, sl] = acc_v[rl, sl] + vm * g_v[e, sl]
            return carry2

        lax.fori_loop(0, CE, e_body, 0)
        return carry

    lax.fori_loop(0, nchunks, chunk_body, 0)
    pltpu.sync_copy(acc_v, out_hbm.at[pl.ds(rbase, RPW)])


def kernel(x, spiral_indices, pool_row, pool_col, pool_val, W, b):
    x2d = x[0]
    w3 = W.reshape(SPIRAL_LEN, IN_C, OUT_C)
    table = _project(x2d, w3)

    idxf = (spiral_indices.astype(jnp.int32)
            + (jnp.arange(SPIRAL_LEN, dtype=jnp.int32)
               * N_NODES)[None, :]).reshape(-1)
    idxf = jnp.concatenate(
        [idxf, jnp.zeros(N_PAD * SPIRAL_LEN - N_NODES * SPIRAL_LEN,
                         jnp.int32)])
    h = _spiral(table, idxf, b)

    rowi = pool_row.astype(jnp.int32)
    bounds = jnp.searchsorted(
        rowi, jnp.arange(NW + 1, dtype=jnp.int32) * RPW).astype(jnp.int32)
    bounds = jnp.concatenate([bounds, jnp.zeros(7, jnp.int32)])

    pad_e = NNZ_PAD - NNZ
    colp = jnp.concatenate([pool_col.astype(jnp.int32),
                            jnp.zeros(pad_e, jnp.int32)])
    valp = jnp.concatenate([pool_val, jnp.zeros(pad_e, jnp.float32)])
    rowp = jnp.concatenate([rowi, jnp.zeros(pad_e, jnp.int32)])

    pooled = _pool(h, colp, valp, rowp, bounds)
    return pooled[:N_DOWN][None]


# speedup vs baseline: 1.3453x; 1.3453x over previous
import functools

import jax
import jax.numpy as jnp
from jax import lax
from jax.experimental import pallas as pl
from jax.experimental.pallas import tpu as pltpu
from jax.experimental.pallas import tpu_sc as plsc

N_NODES = 100000
N_DOWN = 25000
SPIRAL_LEN = 9
IN_C = 64
OUT_C = 64
NNZ = 100000

N_SEC = 102400

NW = 32
NPW = 3128
N_PAD = NW * NPW
CB = 136
RG = CB * SPIRAL_LEN

RPW = 784
OUT_PAD = NW * RPW
CE = 128
NNZ_PAD = NNZ + 2 * CE

_mesh = plsc.VectorSubcoreMesh(core_axis_name="c", subcore_axis_name="s")


def _sget(ref, i):
    return ref[pl.ds(i, 16)][0]


def _mm_body(xt_ref, w_ref, o_ref):
    o_ref[...] = lax.dot_general(
        xt_ref[...], w_ref[0],
        dimension_numbers=(((0,), (0,)), ((), ())),
        preferred_element_type=jnp.float32)


def _project(xt, w3):
    bn = 12800
    nb = N_SEC // bn
    return pl.pallas_call(
        _mm_body,
        grid=(nb, SPIRAL_LEN),
        in_specs=[
            pl.BlockSpec((IN_C, bn), lambda i, l: (0, i)),
            pl.BlockSpec((1, IN_C, OUT_C), lambda i, l: (l, 0, 0)),
        ],
        out_specs=pl.BlockSpec((bn, OUT_C), lambda i, l: (l * nb + i, 0)),
        out_shape=jax.ShapeDtypeStruct((N_SEC * SPIRAL_LEN, OUT_C),
                                       jnp.float32),
    )(xt, w3)


@functools.partial(
    pl.kernel,
    mesh=_mesh,
    compiler_params=pltpu.CompilerParams(use_tc_tiling_on_sc=False),
    out_type=jax.ShapeDtypeStruct((N_PAD, OUT_C), jnp.float32),
    scratch_types=[
        pltpu.VMEM((RG,), jnp.int32),
        pltpu.VMEM((RG, OUT_C), jnp.float32),
        pltpu.VMEM((CB, OUT_C), jnp.float32),
        pltpu.VMEM((OUT_C,), jnp.float32),
        pltpu.SemaphoreType.DMA,
    ],
)
def _spiral(table_hbm, idx_hbm, b_hbm, h_hbm, idx_v, g_v, h_v, b_v, sem):
    wid = lax.axis_index("s") * 2 + lax.axis_index("c")
    base_node = wid * NPW
    pltpu.sync_copy(b_hbm, b_v)

    def chunk_body(ci, carry):
        nbase = base_node + ci * CB
        pltpu.sync_copy(idx_hbm.at[pl.ds(nbase * SPIRAL_LEN, RG)], idx_v)
        cps = []
        for s in range(9):
            cps.append(pltpu.async_copy(
                table_hbm.at[idx_v.at[pl.ds(s * 128, 128)]],
                g_v.at[pl.ds(s * 128, 128)], sem))
        cps.append(pltpu.async_copy(
            table_hbm.at[idx_v.at[pl.ds(1152, 72)]],
            g_v.at[pl.ds(1152, 72)], sem))
        for cp in cps:
            cp.wait()

        def node_body(c, carry2):
            r0 = c * SPIRAL_LEN
            for j in range(4):
                sl = pl.ds(j * 16, 16)
                v = g_v[r0, sl]
                for l in range(1, SPIRAL_LEN):
                    v = v + g_v[r0 + l, sl]
                v = v + b_v[sl]
                v = jnp.where(v > 0.0, v, jnp.exp(v) - 1.0)
                h_v[c, sl] = v
            return carry2

        lax.fori_loop(0, CB, node_body, 0)
        pltpu.sync_copy(h_v, h_hbm.at[pl.ds(nbase, CB)])
        return carry

    lax.fori_loop(0, NPW // CB, chunk_body, 0)


@functools.partial(
    pl.kernel,
    mesh=_mesh,
    compiler_params=pltpu.CompilerParams(use_tc_tiling_on_sc=False),
    out_type=jax.ShapeDtypeStruct((OUT_PAD, OUT_C), jnp.float32),
    scratch_types=[
        pltpu.VMEM((NW + 1 + 16,), jnp.int32),
        pltpu.VMEM((CE + 16,), jnp.int32),
        pltpu.VMEM((CE + 16,), jnp.float32),
        pltpu.VMEM((CE + 16,), jnp.int32),
        pltpu.VMEM((CE, OUT_C), jnp.float32),
        pltpu.VMEM((RPW, OUT_C), jnp.float32),
        pltpu.SemaphoreType.DMA,
    ],
)
def _pool(h_hbm, col_hbm, val_hbm, row_hbm, bnd_hbm, out_hbm,
          bnd_v, col_v, val_v, row_v, g_v, acc_v, sem):
    wid = lax.axis_index("s") * 2 + lax.axis_index("c")
    rbase = wid * RPW
    pltpu.sync_copy(bnd_hbm, bnd_v.at[pl.ds(0, 40)])
    k0 = _sget(bnd_v, wid)
    k1 = _sget(bnd_v, wid + 1)
    k0a = jnp.bitwise_and(k0, -8)
    nchunks = lax.shift_right_logical(k1 - k0a + (CE - 1), 7)

    zero16 = jnp.zeros((16,), jnp.float32)

    def zero_body(r, carry):
        for j in range(4):
            acc_v[r, pl.ds(j * 16, 16)] = zero16
        return carry

    lax.fori_loop(0, RPW, zero_body, 0)

    def chunk_body(ci, carry):
        kc = pl.multiple_of(k0a + ci * CE, 8)
        pltpu.sync_copy(col_hbm.at[pl.ds(kc, CE)], col_v.at[pl.ds(0, CE)])
        pltpu.sync_copy(val_hbm.at[pl.ds(kc, CE)], val_v.at[pl.ds(0, CE)])
        pltpu.sync_copy(row_hbm.at[pl.ds(kc, CE)], row_v.at[pl.ds(0, CE)])
        pltpu.async_copy(h_hbm.at[col_v.at[pl.ds(0, CE)]], g_v, sem).wait()

        def e_body(e, carry2):
            kg = kc + e
            ok = jnp.logical_and(kg >= k0, kg < k1)
            vm = jnp.where(ok, _sget(val_v, e), 0.0)
            rl = jnp.clip(_sget(row_v, e) - rbase, 0, RPW - 1)
            for j in range(4):
                sl = pl.ds(j * 16, 16)
                acc_v[rl, sl] = acc_v[rl, sl] + vm * g_v[e, sl]
            return carry2

        lax.fori_loop(0, CE, e_body, 0)
        return carry

    lax.fori_loop(0, nchunks, chunk_body, 0)
    pltpu.sync_copy(acc_v, out_hbm.at[pl.ds(rbase, RPW)])


def kernel(x, spiral_indices, pool_row, pool_col, pool_val, W, b):
    xt = x[0].T
    w3 = W.reshape(SPIRAL_LEN, IN_C, OUT_C)
    table = _project(xt, w3)

    idxf = (spiral_indices.astype(jnp.int32)
            + (jnp.arange(SPIRAL_LEN, dtype=jnp.int32)
               * N_SEC)[None, :]).reshape(-1)
    idxf = jnp.concatenate(
        [idxf, jnp.zeros(N_PAD * SPIRAL_LEN - N_NODES * SPIRAL_LEN,
                         jnp.int32)])
    h = _spiral(table, idxf, b)

    rowi = pool_row.astype(jnp.int32)
    bounds = jnp.searchsorted(
        rowi, jnp.arange(NW + 1, dtype=jnp.int32) * RPW).astype(jnp.int32)
    bounds = jnp.concatenate([bounds, jnp.zeros(7, jnp.int32)])

    pad_e = NNZ_PAD - NNZ
    colp = jnp.concatenate([pool_col.astype(jnp.int32),
                            jnp.zeros(pad_e, jnp.int32)])
    valp = jnp.concatenate([pool_val, jnp.zeros(pad_e, jnp.float32)])
    rowp = jnp.concatenate([rowi, jnp.zeros(pad_e, jnp.int32)])

    pooled = _pool(h, colp, valp, rowp, bounds)
    return pooled[:N_DOWN][None]


# speedup vs baseline: 1.8769x; 1.3951x over previous
import functools

import jax
import jax.numpy as jnp
from jax import lax
from jax.experimental import pallas as pl
from jax.experimental.pallas import tpu as pltpu
from jax.experimental.pallas import tpu_sc as plsc

N_NODES = 100000
N_DOWN = 25000
SPIRAL_LEN = 9
IN_C = 64
OUT_C = 64
NNZ = 100000

N_SEC = 102400

NW = 32
NPW = 3128
N_PAD = NW * NPW
CB = 136
RG = CB * SPIRAL_LEN

RPW = 784
OUT_PAD = NW * RPW
CE = 128
NNZ_PAD = NNZ + 2 * CE

_mesh = plsc.VectorSubcoreMesh(core_axis_name="c", subcore_axis_name="s")


def _sget(ref, i):
    return ref[pl.ds(i, 16)][0]


def _mm_body(xt_ref, w_ref, o_ref):
    o_ref[...] = lax.dot_general(
        xt_ref[...], w_ref[0],
        dimension_numbers=(((0,), (0,)), ((), ())),
        preferred_element_type=jnp.float32)


def _project(xt, w3):
    bn = 12800
    nb = N_SEC // bn
    return pl.pallas_call(
        _mm_body,
        grid=(nb, SPIRAL_LEN),
        in_specs=[
            pl.BlockSpec((IN_C, bn), lambda i, l: (0, i)),
            pl.BlockSpec((1, IN_C, 2 * OUT_C), lambda i, l: (l, 0, 0)),
        ],
        out_specs=pl.BlockSpec((bn, 2 * OUT_C), lambda i, l: (l * nb + i, 0)),
        out_shape=jax.ShapeDtypeStruct((N_SEC * SPIRAL_LEN, 2 * OUT_C),
                                       jnp.float32),
    )(xt, w3)


@functools.partial(
    pl.kernel,
    mesh=_mesh,
    compiler_params=pltpu.CompilerParams(use_tc_tiling_on_sc=False),
    out_type=jax.ShapeDtypeStruct((N_PAD, OUT_C), jnp.float32),
    scratch_types=[
        pltpu.VMEM((RG,), jnp.int32),
        pltpu.VMEM((RG, OUT_C), jnp.float32),
        pltpu.VMEM((CB, OUT_C), jnp.float32),
        pltpu.VMEM((OUT_C,), jnp.float32),
        pltpu.SemaphoreType.DMA,
    ],
)
def _spiral(table_hbm, idx_hbm, b_hbm, h_hbm, idx_v, g_v, h_v, b_v, sem):
    wid = lax.axis_index("s") * 2 + lax.axis_index("c")
    base_node = wid * NPW
    pltpu.sync_copy(b_hbm, b_v)

    def chunk_body(ci, carry):
        nbase = base_node + ci * CB
        pltpu.sync_copy(idx_hbm.at[pl.ds(nbase * SPIRAL_LEN, RG)], idx_v)
        cps = []
        for s in range(9):
            cps.append(pltpu.async_copy(
                table_hbm.at[idx_v.at[pl.ds(s * 128, 128)]],
                g_v.at[pl.ds(s * 128, 128)], sem))
        cps.append(pltpu.async_copy(
            table_hbm.at[idx_v.at[pl.ds(1152, 72)]],
            g_v.at[pl.ds(1152, 72)], sem))
        for cp in cps:
            cp.wait()

        def node_body(c, carry2):
            r0 = c * SPIRAL_LEN
            for j in range(4):
                sl = pl.ds(j * 16, 16)
                v = g_v[r0, sl]
                for l in range(1, SPIRAL_LEN):
                    v = v + g_v[r0 + l, sl]
                v = v + b_v[sl]
                v = jnp.where(v > 0.0, v, jnp.exp(v) - 1.0)
                h_v[c, sl] = v
            return carry2

        lax.fori_loop(0, CB, node_body, 0)
        pltpu.sync_copy(h_v, h_hbm.at[pl.ds(nbase, CB)])
        return carry

    lax.fori_loop(0, NPW // CB, chunk_body, 0)


@functools.partial(
    pl.kernel,
    mesh=_mesh,
    compiler_params=pltpu.CompilerParams(use_tc_tiling_on_sc=False),
    out_type=jax.ShapeDtypeStruct((OUT_PAD, OUT_C), jnp.float32),
    scratch_types=[
        pltpu.VMEM((NW + 1 + 16,), jnp.int32),
        pltpu.VMEM((CE + 16,), jnp.int32),
        pltpu.VMEM((CE + 16,), jnp.float32),
        pltpu.VMEM((CE + 16,), jnp.int32),
        pltpu.VMEM((CE, OUT_C), jnp.float32),
        pltpu.VMEM((RPW, OUT_C), jnp.float32),
        pltpu.SemaphoreType.DMA,
    ],
)
def _pool(h_hbm, col_hbm, val_hbm, row_hbm, bnd_hbm, out_hbm,
          bnd_v, col_v, val_v, row_v, g_v, acc_v, sem):
    wid = lax.axis_index("s") * 2 + lax.axis_index("c")
    rbase = wid * RPW
    pltpu.sync_copy(bnd_hbm, bnd_v.at[pl.ds(0, 40)])
    k0 = _sget(bnd_v, wid)
    k1 = _sget(bnd_v, wid + 1)
    k0a = jnp.bitwise_and(k0, -8)
    nchunks = lax.shift_right_logical(k1 - k0a + (CE - 1), 7)

    zero16 = jnp.zeros((16,), jnp.float32)

    def zero_body(r, carry):
        for j in range(4):
            acc_v[r, pl.ds(j * 16, 16)] = zero16
        return carry

    lax.fori_loop(0, RPW, zero_body, 0)

    def chunk_body(ci, carry):
        kc = pl.multiple_of(k0a + ci * CE, 8)
        pltpu.sync_copy(col_hbm.at[pl.ds(kc, CE)], col_v.at[pl.ds(0, CE)])
        pltpu.sync_copy(val_hbm.at[pl.ds(kc, CE)], val_v.at[pl.ds(0, CE)])
        pltpu.sync_copy(row_hbm.at[pl.ds(kc, CE)], row_v.at[pl.ds(0, CE)])
        pltpu.async_copy(h_hbm.at[col_v.at[pl.ds(0, CE)]], g_v, sem).wait()

        def e_body(e, carry2):
            kg = kc + e
            ok = jnp.logical_and(kg >= k0, kg < k1)
            vm = jnp.where(ok, _sget(val_v, e), 0.0)
            rl = jnp.clip(_sget(row_v, e) - rbase, 0, RPW - 1)
            for j in range(4):
                sl = pl.ds(j * 16, 16)
                acc_v[rl, sl] = acc_v[rl, sl] + vm * g_v[e, sl]
            return carry2

        lax.fori_loop(0, CE, e_body, 0)
        return carry

    lax.fori_loop(0, nchunks, chunk_body, 0)
    pltpu.sync_copy(acc_v, out_hbm.at[pl.ds(rbase, RPW)])


def kernel(x, spiral_indices, pool_row, pool_col, pool_val, W, b):
    xt = x[0].T
    w3 = jnp.concatenate(
        [W.reshape(SPIRAL_LEN, IN_C, OUT_C),
         jnp.zeros((SPIRAL_LEN, IN_C, OUT_C), jnp.float32)], axis=2)
    table = _project(xt, w3).reshape(N_SEC * SPIRAL_LEN * 2, OUT_C)

    sflat = spiral_indices.astype(jnp.int32).reshape(-1)
    slot = jnp.arange(N_NODES * SPIRAL_LEN, dtype=jnp.int32) % SPIRAL_LEN
    idxf = (sflat + slot * N_SEC) * 2
    idxf = jnp.concatenate(
        [idxf, jnp.zeros(N_PAD * SPIRAL_LEN - N_NODES * SPIRAL_LEN,
                         jnp.int32)])
    h = _spiral(table, idxf, b)

    rowi = pool_row.astype(jnp.int32)
    bounds = jnp.searchsorted(
        rowi, jnp.arange(NW + 1, dtype=jnp.int32) * RPW).astype(jnp.int32)
    bounds = jnp.concatenate([bounds, jnp.zeros(7, jnp.int32)])

    pad_e = NNZ_PAD - NNZ
    colp = jnp.concatenate([pool_col.astype(jnp.int32),
                            jnp.zeros(pad_e, jnp.int32)])
    valp = jnp.concatenate([pool_val, jnp.zeros(pad_e, jnp.float32)])
    rowp = jnp.concatenate([rowi, jnp.zeros(pad_e, jnp.int32)])

    pooled = _pool(h, colp, valp, rowp, bounds)
    return pooled[:N_DOWN][None]


# speedup vs baseline: 2.0366x; 1.0851x over previous
import functools

import jax
import jax.numpy as jnp
from jax import lax
from jax.experimental import pallas as pl
from jax.experimental.pallas import tpu as pltpu
from jax.experimental.pallas import tpu_sc as plsc

N_NODES = 100000
N_DOWN = 25000
SPIRAL_LEN = 9
IN_C = 64
OUT_C = 64
NNZ = 100000

N_SEC = 102400

NW = 32
NPW = 3136
N_PAD = NW * NPW
CB = 56
RG = CB * SPIRAL_LEN

RPW = 784
OUT_PAD = NW * RPW
CE = 128
NNZ_PAD = NNZ + 2 * CE

_mesh = plsc.VectorSubcoreMesh(core_axis_name="c", subcore_axis_name="s")


def _sget(ref, i):
    return ref[pl.ds(i, 16)][0]


def _mm_body(xt_ref, w_ref, o_ref):
    o_ref[...] = lax.dot_general(
        xt_ref[...], w_ref[0],
        dimension_numbers=(((0,), (0,)), ((), ())),
        preferred_element_type=jnp.float32)


def _project(xt, w3):
    bn = 12800
    nb = N_SEC // bn
    return pl.pallas_call(
        _mm_body,
        grid=(nb, SPIRAL_LEN),
        in_specs=[
            pl.BlockSpec((IN_C, bn), lambda i, l: (0, i)),
            pl.BlockSpec((1, IN_C, 2 * OUT_C), lambda i, l: (l, 0, 0)),
        ],
        out_specs=pl.BlockSpec((bn, 2 * OUT_C), lambda i, l: (l * nb + i, 0)),
        out_shape=jax.ShapeDtypeStruct((N_SEC * SPIRAL_LEN, 2 * OUT_C),
                                       jnp.float32),
    )(xt, w3)


def _fire_gathers(table_hbm, idx_v, g_v, par, sem):
    for s in range(0, RG, 128):
        n = min(128, RG - s)
        pltpu.make_async_copy(
            table_hbm.at[idx_v.at[par, pl.ds(s, n)]],
            g_v.at[par, pl.ds(s, n)], sem).start()


@functools.partial(
    pl.kernel,
    mesh=_mesh,
    compiler_params=pltpu.CompilerParams(use_tc_tiling_on_sc=False),
    out_type=jax.ShapeDtypeStruct((N_PAD, OUT_C), jnp.float32),
    scratch_types=[
        pltpu.VMEM((2, RG), jnp.int32),
        pltpu.VMEM((2, RG, OUT_C), jnp.float32),
        pltpu.VMEM((CB, OUT_C), jnp.float32),
        pltpu.VMEM((OUT_C,), jnp.float32),
        pltpu.SemaphoreType.DMA,
        pltpu.SemaphoreType.DMA,
    ],
)
def _spiral(table_hbm, idx_hbm, b_hbm, h_hbm, idx_v, g_v, h_v, b_v,
            sem0, sem1):
    wid = lax.axis_index("s") * 2 + lax.axis_index("c")
    base_node = wid * NPW
    pltpu.sync_copy(b_hbm, b_v)
    nchunks = NPW // CB
    sems = (sem0, sem1)

    pltpu.sync_copy(idx_hbm.at[pl.ds(base_node * SPIRAL_LEN, RG)],
                    idx_v.at[0])
    _fire_gathers(table_hbm, idx_v, g_v, 0, sems[0])

    def chunk_step(ci, par):
        nbase = base_node + ci * CB

        @pl.when(ci + 1 < nchunks)
        def _prefetch():
            pltpu.sync_copy(
                idx_hbm.at[pl.ds((nbase + CB) * SPIRAL_LEN, RG)],
                idx_v.at[1 - par])
            _fire_gathers(table_hbm, idx_v, g_v, 1 - par, sems[1 - par])

        pltpu.make_async_copy(
            table_hbm.at[pl.ds(0, RG)], g_v.at[par], sems[par]).wait()

        def node_body(c, carry2):
            r0 = c * SPIRAL_LEN
            for j in range(4):
                sl = pl.ds(j * 16, 16)
                v = g_v[par, r0, sl]
                for l in range(1, SPIRAL_LEN):
                    v = v + g_v[par, r0 + l, sl]
                v = v + b_v[sl]
                v = jnp.where(v > 0.0, v, jnp.exp(v) - 1.0)
                h_v[c, sl] = v
            return carry2

        lax.fori_loop(0, CB, node_body, 0)
        pltpu.sync_copy(h_v, h_hbm.at[pl.ds(nbase, CB)])

    def pair_body(i, carry):
        chunk_step(2 * i, 0)
        chunk_step(2 * i + 1, 1)
        return carry

    lax.fori_loop(0, nchunks // 2, pair_body, 0)
    if nchunks % 2:
        chunk_step(nchunks - 1, 0)


@functools.partial(
    pl.kernel,
    mesh=_mesh,
    compiler_params=pltpu.CompilerParams(use_tc_tiling_on_sc=False),
    out_type=jax.ShapeDtypeStruct((OUT_PAD, OUT_C), jnp.float32),
    scratch_types=[
        pltpu.VMEM((NW + 1 + 16,), jnp.int32),
        pltpu.VMEM((CE + 16,), jnp.int32),
        pltpu.VMEM((CE + 16,), jnp.float32),
        pltpu.VMEM((CE + 16,), jnp.int32),
        pltpu.VMEM((CE, OUT_C), jnp.float32),
        pltpu.VMEM((RPW, OUT_C), jnp.float32),
        pltpu.SemaphoreType.DMA,
    ],
)
def _pool(h_hbm, col_hbm, val_hbm, row_hbm, bnd_hbm, out_hbm,
          bnd_v, col_v, val_v, row_v, g_v, acc_v, sem):
    wid = lax.axis_index("s") * 2 + lax.axis_index("c")
    rbase = wid * RPW
    pltpu.sync_copy(bnd_hbm, bnd_v.at[pl.ds(0, 40)])
    k0 = _sget(bnd_v, wid)
    k1 = _sget(bnd_v, wid + 1)
    k0a = jnp.bitwise_and(k0, -8)
    nchunks = lax.shift_right_logical(k1 - k0a + (CE - 1), 7)

    zero16 = jnp.zeros((16,), jnp.float32)

    def zero_body(r, carry):
        for j in range(4):
            acc_v[r, pl.ds(j * 16, 16)] = zero16
        return carry

    lax.fori_loop(0, RPW, zero_body, 0)

    def chunk_body(ci, carry):
        kc = pl.multiple_of(k0a + ci * CE, 8)
        pltpu.sync_copy(col_hbm.at[pl.ds(kc, CE)], col_v.at[pl.ds(0, CE)])
        pltpu.sync_copy(val_hbm.at[pl.ds(kc, CE)], val_v.at[pl.ds(0, CE)])
        pltpu.sync_copy(row_hbm.at[pl.ds(kc, CE)], row_v.at[pl.ds(0, CE)])
        pltpu.async_copy(h_hbm.at[col_v.at[pl.ds(0, CE)]], g_v, sem).wait()

        iota16 = lax.iota(jnp.int32, 16)

        def grp_body(g, carry2):
            e0 = g * 16
            kg = kc + e0 + iota16
            ok = jnp.logical_and(kg >= k0, kg < k1)
            vm16 = jnp.where(ok, val_v[pl.ds(e0, 16)], 0.0)
            r16 = jnp.clip(row_v[pl.ds(e0, 16)] - rbase, 0, RPW - 1)
            for j in range(16):
                rl = r16[j]
                vm = vm16[j]
                for q in range(4):
                    sl = pl.ds(q * 16, 16)
                    acc_v[rl, sl] = acc_v[rl, sl] + vm * g_v[e0 + j, sl]
            return carry2

        lax.fori_loop(0, CE // 16, grp_body, 0)
        return carry

    lax.fori_loop(0, nchunks, chunk_body, 0)
    pltpu.sync_copy(acc_v, out_hbm.at[pl.ds(rbase, RPW)])


def kernel(x, spiral_indices, pool_row, pool_col, pool_val, W, b):
    xt = x[0].T
    w3 = jnp.concatenate(
        [W.reshape(SPIRAL_LEN, IN_C, OUT_C),
         jnp.zeros((SPIRAL_LEN, IN_C, OUT_C), jnp.float32)], axis=2)
    table = _project(xt, w3).reshape(N_SEC * SPIRAL_LEN * 2, OUT_C)

    sflat = spiral_indices.astype(jnp.int32).reshape(-1)
    slot = jnp.arange(N_NODES * SPIRAL_LEN, dtype=jnp.int32) % SPIRAL_LEN
    idxf = (sflat + slot * N_SEC) * 2
    idxf = jnp.concatenate(
        [idxf, jnp.zeros(N_PAD * SPIRAL_LEN - N_NODES * SPIRAL_LEN,
                         jnp.int32)])
    h = _spiral(table, idxf, b)

    rowi = pool_row.astype(jnp.int32)
    bounds = jnp.searchsorted(
        rowi, jnp.arange(NW + 1, dtype=jnp.int32) * RPW).astype(jnp.int32)
    bounds = jnp.concatenate([bounds, jnp.zeros(7, jnp.int32)])

    pad_e = NNZ_PAD - NNZ
    colp = jnp.concatenate([pool_col.astype(jnp.int32),
                            jnp.zeros(pad_e, jnp.int32)])
    valp = jnp.concatenate([pool_val, jnp.zeros(pad_e, jnp.float32)])
    rowp = jnp.concatenate([rowi, jnp.zeros(pad_e, jnp.int32)])

    pooled = _pool(h, colp, valp, rowp, bounds)
    return pooled[:N_DOWN][None]


# speedup vs baseline: 2.5884x; 1.2709x over previous
import functools

import jax
import jax.numpy as jnp
from jax import lax
from jax.experimental import pallas as pl
from jax.experimental.pallas import tpu as pltpu
from jax.experimental.pallas import tpu_sc as plsc

N_NODES = 100000
N_DOWN = 25000
SPIRAL_LEN = 9
IN_C = 64
OUT_C = 64
NNZ = 100000

N_SEC = 102400

NW = 32
NPW = 3136
N_PAD = NW * NPW
CB = 112

RPW = 784
OUT_PAD = NW * RPW
CE = 128
NNZ_PAD = NNZ + 2 * CE

_mesh = plsc.VectorSubcoreMesh(core_axis_name="c", subcore_axis_name="s")


def _sget(ref, i):
    return ref[pl.ds(i, 16)][0]


def _mm_body(xt_ref, w_ref, o_ref):
    o_ref[...] = lax.dot_general(
        xt_ref[...], w_ref[0],
        dimension_numbers=(((0,), (0,)), ((), ())),
        preferred_element_type=jnp.float32)


def _project(xt, w3):
    bn = 12800
    nb = N_SEC // bn
    return pl.pallas_call(
        _mm_body,
        grid=(nb, SPIRAL_LEN),
        in_specs=[
            pl.BlockSpec((IN_C, bn), lambda i, l: (0, i)),
            pl.BlockSpec((1, IN_C, 2 * OUT_C), lambda i, l: (l, 0, 0)),
        ],
        out_specs=pl.BlockSpec((bn, 2 * OUT_C), lambda i, l: (l * nb + i, 0)),
        out_shape=jax.ShapeDtypeStruct((N_SEC * SPIRAL_LEN, 2 * OUT_C),
                                       jnp.float32),
    )(xt, w3)


def _fire_gathers(table_hbm, idx_v, h_v, par, sem):
    for l in range(SPIRAL_LEN):
        pltpu.async_copy(
            table_hbm.at[idx_v.at[par, l]],
            h_v.at[par], sem, add=True)


@functools.partial(
    pl.kernel,
    mesh=_mesh,
    compiler_params=pltpu.CompilerParams(use_tc_tiling_on_sc=False),
    out_type=jax.ShapeDtypeStruct((N_PAD, OUT_C), jnp.float32),
    scratch_types=[
        pltpu.VMEM((2, SPIRAL_LEN, CB), jnp.int32),
        pltpu.VMEM((2, CB, OUT_C), jnp.float32),
        pltpu.VMEM((OUT_C,), jnp.float32),
        pltpu.SemaphoreType.DMA,
        pltpu.SemaphoreType.DMA,
    ],
)
def _spiral(table_hbm, idxt_hbm, b_hbm, h_hbm, idx_v, h_v, b_v,
            sem0, sem1):
    wid = lax.axis_index("s") * 2 + lax.axis_index("c")
    base_node = wid * NPW
    pltpu.sync_copy(b_hbm, b_v)
    nchunks = NPW // CB
    sems = (sem0, sem1)

    def zero_buf(par):
        def zb(c, carry):
            for j in range(4):
                h_v[par, c, pl.ds(j * 16, 16)] = jnp.zeros((16,),
                                                           jnp.float32)
            return carry
        lax.fori_loop(0, CB, zb, 0)

    def prefetch(nbase, par):
        zero_buf(par)
        pltpu.sync_copy(idxt_hbm.at[:, pl.ds(nbase, CB)], idx_v.at[par])
        _fire_gathers(table_hbm, idx_v, h_v, par, sems[par])

    prefetch(base_node, 0)

    def chunk_step(ci, par):
        nbase = base_node + ci * CB

        @pl.when(ci + 1 < nchunks)
        def _pf():
            prefetch(nbase + CB, 1 - par)

        for _ in range(SPIRAL_LEN):
            pltpu.make_async_copy(
                table_hbm.at[pl.ds(0, CB)], h_v.at[par], sems[par]).wait()

        def node_body(c, carry2):
            for j in range(4):
                sl = pl.ds(j * 16, 16)
                v = h_v[par, c, sl] + b_v[sl]
                v = jnp.where(v > 0.0, v, jnp.exp(v) - 1.0)
                h_v[par, c, sl] = v
            return carry2

        lax.fori_loop(0, CB, node_body, 0)
        pltpu.sync_copy(h_v.at[par], h_hbm.at[pl.ds(nbase, CB)])

    def pair_body(i, carry):
        chunk_step(2 * i, 0)
        chunk_step(2 * i + 1, 1)
        return carry

    lax.fori_loop(0, nchunks // 2, pair_body, 0)
    if nchunks % 2:
        chunk_step(nchunks - 1, 0)


@functools.partial(
    pl.kernel,
    mesh=_mesh,
    compiler_params=pltpu.CompilerParams(use_tc_tiling_on_sc=False),
    out_type=jax.ShapeDtypeStruct((OUT_PAD, OUT_C), jnp.float32),
    scratch_types=[
        pltpu.VMEM((NW + 1 + 16,), jnp.int32),
        pltpu.VMEM((CE + 16,), jnp.int32),
        pltpu.VMEM((CE + 16,), jnp.float32),
        pltpu.VMEM((CE + 16,), jnp.int32),
        pltpu.VMEM((CE, OUT_C), jnp.float32),
        pltpu.VMEM((RPW, OUT_C), jnp.float32),
        pltpu.SemaphoreType.DMA,
    ],
)
def _pool(h_hbm, col_hbm, val_hbm, row_hbm, bnd_hbm, out_hbm,
          bnd_v, col_v, val_v, row_v, g_v, acc_v, sem):
    wid = lax.axis_index("s") * 2 + lax.axis_index("c")
    rbase = wid * RPW
    pltpu.sync_copy(bnd_hbm, bnd_v.at[pl.ds(0, 40)])
    k0 = _sget(bnd_v, wid)
    k1 = _sget(bnd_v, wid + 1)
    k0a = jnp.bitwise_and(k0, -8)
    nchunks = lax.shift_right_logical(k1 - k0a + (CE - 1), 7)

    zero16 = jnp.zeros((16,), jnp.float32)

    def zero_body(r, carry):
        for j in range(4):
            acc_v[r, pl.ds(j * 16, 16)] = zero16
        return carry

    lax.fori_loop(0, RPW, zero_body, 0)

    def chunk_body(ci, carry):
        kc = pl.multiple_of(k0a + ci * CE, 8)
        pltpu.sync_copy(col_hbm.at[pl.ds(kc, CE)], col_v.at[pl.ds(0, CE)])
        pltpu.sync_copy(val_hbm.at[pl.ds(kc, CE)], val_v.at[pl.ds(0, CE)])
        pltpu.sync_copy(row_hbm.at[pl.ds(kc, CE)], row_v.at[pl.ds(0, CE)])
        pltpu.async_copy(h_hbm.at[col_v.at[pl.ds(0, CE)]], g_v, sem).wait()

        iota16 = lax.iota(jnp.int32, 16)

        def grp_body(g, carry2):
            e0 = g * 16
            kg = kc + e0 + iota16
            ok = jnp.logical_and(kg >= k0, kg < k1)
            vm16 = jnp.where(ok, val_v[pl.ds(e0, 16)], 0.0)
            r16 = jnp.clip(row_v[pl.ds(e0, 16)] - rbase, 0, RPW - 1)
            for j in range(16):
                rl = r16[j]
                vm = vm16[j]
                for q in range(4):
                    sl = pl.ds(q * 16, 16)
                    acc_v[rl, sl] = acc_v[rl, sl] + vm * g_v[e0 + j, sl]
            return carry2

        lax.fori_loop(0, CE // 16, grp_body, 0)
        return carry

    lax.fori_loop(0, nchunks, chunk_body, 0)
    pltpu.sync_copy(acc_v, out_hbm.at[pl.ds(rbase, RPW)])


def kernel(x, spiral_indices, pool_row, pool_col, pool_val, W, b):
    xt = x[0].T
    w3 = jnp.concatenate(
        [W.reshape(SPIRAL_LEN, IN_C, OUT_C),
         jnp.zeros((SPIRAL_LEN, IN_C, OUT_C), jnp.float32)], axis=2)
    table = _project(xt, w3).reshape(N_SEC * SPIRAL_LEN * 2, OUT_C)

    idxt = (spiral_indices.astype(jnp.int32).T
            + jnp.arange(SPIRAL_LEN, dtype=jnp.int32)[:, None] * N_SEC) * 2
    idxt = jnp.pad(idxt, ((0, 0), (0, N_PAD - N_NODES)))
    h = _spiral(table, idxt, b)

    rowi = pool_row.astype(jnp.int32)
    bounds = jnp.searchsorted(
        rowi, jnp.arange(NW + 1, dtype=jnp.int32) * RPW).astype(jnp.int32)
    bounds = jnp.concatenate([bounds, jnp.zeros(7, jnp.int32)])

    pad_e = NNZ_PAD - NNZ
    colp = jnp.concatenate([pool_col.astype(jnp.int32),
                            jnp.zeros(pad_e, jnp.int32)])
    valp = jnp.concatenate([pool_val, jnp.zeros(pad_e, jnp.float32)])
    rowp = jnp.concatenate([rowi, jnp.zeros(pad_e, jnp.int32)])

    pooled = _pool(h, colp, valp, rowp, bounds)
    return pooled[:N_DOWN][None]


# speedup vs baseline: 2.7218x; 1.0516x over previous
import functools

import jax
import jax.numpy as jnp
from jax import lax
from jax.experimental import pallas as pl
from jax.experimental.pallas import tpu as pltpu
from jax.experimental.pallas import tpu_sc as plsc

N_NODES = 100000
N_DOWN = 25000
SPIRAL_LEN = 9
IN_C = 64
OUT_C = 64
NNZ = 100000

N_SEC = 102400

NW = 32
NPW = 3136
N_PAD = NW * NPW
CB = 112

RPW = 784
OUT_PAD = NW * RPW
CE = 128
NNZ_PAD = NNZ + 2 * CE

_mesh = plsc.VectorSubcoreMesh(core_axis_name="c", subcore_axis_name="s")


def _sget(ref, i):
    return ref[pl.ds(i, 16)][0]


def _mm_body(xt_ref, w_ref, o_ref):
    o_ref[...] = lax.dot_general(
        xt_ref[...], w_ref[0],
        dimension_numbers=(((0,), (0,)), ((), ())),
        preferred_element_type=jnp.float32)


def _project(xt, w3):
    bn = 12800
    nb = N_SEC // bn
    return pl.pallas_call(
        _mm_body,
        grid=(nb, SPIRAL_LEN),
        in_specs=[
            pl.BlockSpec((IN_C, bn), lambda i, l: (0, i)),
            pl.BlockSpec((1, IN_C, 2 * OUT_C), lambda i, l: (l, 0, 0)),
        ],
        out_specs=pl.BlockSpec((bn, 2 * OUT_C), lambda i, l: (l * nb + i, 0)),
        out_shape=jax.ShapeDtypeStruct((N_SEC * SPIRAL_LEN, 2 * OUT_C),
                                       jnp.float32),
    )(xt, w3)


def _fire_gathers(table_hbm, idx_v, h_v, par, sem):
    for l in range(SPIRAL_LEN):
        pltpu.async_copy(
            table_hbm.at[idx_v.at[par, l]],
            h_v.at[par], sem, add=True)


@functools.partial(
    pl.kernel,
    mesh=_mesh,
    compiler_params=pltpu.CompilerParams(use_tc_tiling_on_sc=False),
    out_type=jax.ShapeDtypeStruct((N_PAD, OUT_C), jnp.float32),
    scratch_types=[
        pltpu.VMEM((2, SPIRAL_LEN, CB), jnp.int32),
        pltpu.VMEM((2, CB, OUT_C), jnp.float32),
        pltpu.VMEM((OUT_C,), jnp.float32),
        pltpu.SemaphoreType.DMA,
        pltpu.SemaphoreType.DMA,
    ],
)
def _spiral(table_hbm, idxt_hbm, b_hbm, h_hbm, idx_v, h_v, b_v,
            sem0, sem1):
    wid = lax.axis_index("s") * 2 + lax.axis_index("c")
    base_node = wid * NPW
    pltpu.sync_copy(b_hbm, b_v)
    nchunks = NPW // CB
    sems = (sem0, sem1)

    def zero_buf(par):
        def zb(c, carry):
            for j in range(4):
                h_v[par, c, pl.ds(j * 16, 16)] = jnp.zeros((16,),
                                                           jnp.float32)
            return carry
        lax.fori_loop(0, CB, zb, 0)

    def prefetch(nbase, par):
        zero_buf(par)
        pltpu.sync_copy(idxt_hbm.at[:, pl.ds(nbase, CB)], idx_v.at[par])
        _fire_gathers(table_hbm, idx_v, h_v, par, sems[par])

    prefetch(base_node, 0)

    def chunk_step(ci, par):
        nbase = base_node + ci * CB

        @pl.when(ci + 1 < nchunks)
        def _pf():
            prefetch(nbase + CB, 1 - par)

        for _ in range(SPIRAL_LEN):
            pltpu.make_async_copy(
                table_hbm.at[pl.ds(0, CB)], h_v.at[par], sems[par]).wait()

        def node_body(c, carry2):
            for j in range(4):
                sl = pl.ds(j * 16, 16)
                v = h_v[par, c, sl] + b_v[sl]
                v = jnp.where(v > 0.0, v, jnp.exp(v) - 1.0)
                h_v[par, c, sl] = v
            return carry2

        lax.fori_loop(0, CB, node_body, 0)
        pltpu.sync_copy(h_v.at[par], h_hbm.at[pl.ds(nbase, CB)])

    def pair_body(i, carry):
        chunk_step(2 * i, 0)
        chunk_step(2 * i + 1, 1)
        return carry

    lax.fori_loop(0, nchunks // 2, pair_body, 0)
    if nchunks % 2:
        chunk_step(nchunks - 1, 0)


@functools.partial(
    pl.kernel,
    mesh=_mesh,
    compiler_params=pltpu.CompilerParams(use_tc_tiling_on_sc=False),
    out_type=jax.ShapeDtypeStruct((OUT_PAD, OUT_C), jnp.float32),
    scratch_types=[
        pltpu.VMEM((NW + 1 + 16,), jnp.int32),
        pltpu.VMEM((2, CE + 16), jnp.int32),
        pltpu.VMEM((2, CE + 16), jnp.float32),
        pltpu.VMEM((2, CE + 16), jnp.int32),
        pltpu.VMEM((2, CE, OUT_C), jnp.float32),
        pltpu.VMEM((RPW, OUT_C), jnp.float32),
        pltpu.SemaphoreType.DMA,
        pltpu.SemaphoreType.DMA,
        pltpu.SemaphoreType.DMA,
        pltpu.SemaphoreType.DMA,
    ],
)
def _pool(h_hbm, col_hbm, val_hbm, row_hbm, bnd_hbm, out_hbm,
          bnd_v, col_v, val_v, row_v, g_v, acc_v,
          gsem0, gsem1, msem0, msem1):
    wid = lax.axis_index("s") * 2 + lax.axis_index("c")
    rbase = wid * RPW
    pltpu.sync_copy(bnd_hbm, bnd_v.at[pl.ds(0, 40)])
    k0 = _sget(bnd_v, wid)
    k1 = _sget(bnd_v, wid + 1)
    k0a = jnp.bitwise_and(k0, -8)
    nchunks = lax.shift_right_logical(k1 - k0a + (CE - 1), 7)
    gsems = (gsem0, gsem1)
    msems = (msem0, msem1)

    def fire_meta(kc, par):
        for src, dst in ((col_hbm, col_v), (val_hbm, val_v),
                         (row_hbm, row_v)):
            pltpu.make_async_copy(src.at[pl.ds(kc, CE)],
                                  dst.at[par, pl.ds(0, CE)],
                                  msems[par]).start()

    def drain_meta(par):
        for dst in (col_v, val_v, row_v):
            pltpu.make_async_copy(col_hbm.at[pl.ds(0, CE)],
                                  dst.at[par, pl.ds(0, CE)],
                                  msems[par]).wait()

    def fire_gather(par):
        pltpu.make_async_copy(h_hbm.at[col_v.at[par, pl.ds(0, CE)]],
                              g_v.at[par], gsems[par]).start()

    def drain_gather(par):
        pltpu.make_async_copy(h_hbm.at[pl.ds(0, CE)], g_v.at[par],
                              gsems[par]).wait()

    zero16 = jnp.zeros((16,), jnp.float32)

    def zero_body(r, carry):
        for j in range(4):
            acc_v[r, pl.ds(j * 16, 16)] = zero16
        return carry

    @pl.when(nchunks > 0)
    def _prologue():
        fire_meta(pl.multiple_of(k0a, 8), 0)
        drain_meta(0)
        fire_gather(0)

    lax.fori_loop(0, RPW, zero_body, 0)

    iota16 = lax.iota(jnp.int32, 16)

    def chunk_step(ci, par):
        @pl.when(ci < nchunks)
        def _step():
            kc = pl.multiple_of(k0a + ci * CE, 8)

            @pl.when(ci + 1 < nchunks)
            def _fm():
                fire_meta(pl.multiple_of(kc + CE, 8), 1 - par)

            drain_gather(par)

            def grp_body(g, carry2):
                e0 = g * 16
                kg = kc + e0 + iota16
                ok = jnp.logical_and(kg >= k0, kg < k1)
                vm16 = jnp.where(ok, val_v[par, pl.ds(e0, 16)], 0.0)
                r16 = jnp.clip(row_v[par, pl.ds(e0, 16)] - rbase,
                               0, RPW - 1)
                for j in range(16):
                    rl = r16[j]
                    vm = vm16[j]
                    for q in range(4):
                        sl = pl.ds(q * 16, 16)
                        acc_v[rl, sl] = (acc_v[rl, sl]
                                         + vm * g_v[par, e0 + j, sl])
                return carry2

            lax.fori_loop(0, CE // 16, grp_body, 0)

            @pl.when(ci + 1 < nchunks)
            def _fg():
                drain_meta(1 - par)
                fire_gather(1 - par)

    def pair_body(i, carry):
        chunk_step(2 * i, 0)
        chunk_step(2 * i + 1, 1)
        return carry

    lax.fori_loop(0, lax.shift_right_logical(nchunks + 1, 1), pair_body, 0)
    pltpu.sync_copy(acc_v, out_hbm.at[pl.ds(rbase, RPW)])


def kernel(x, spiral_indices, pool_row, pool_col, pool_val, W, b):
    xt = x[0].T
    w3 = jnp.concatenate(
        [W.reshape(SPIRAL_LEN, IN_C, OUT_C),
         jnp.zeros((SPIRAL_LEN, IN_C, OUT_C), jnp.float32)], axis=2)
    table = _project(xt, w3).reshape(N_SEC * SPIRAL_LEN * 2, OUT_C)

    idxt = (spiral_indices.astype(jnp.int32).T
            + jnp.arange(SPIRAL_LEN, dtype=jnp.int32)[:, None] * N_SEC) * 2
    idxt = jnp.pad(idxt, ((0, 0), (0, N_PAD - N_NODES)))
    h = _spiral(table, idxt, b)

    rowi = pool_row.astype(jnp.int32)
    bounds = jnp.searchsorted(
        rowi, jnp.arange(NW + 1, dtype=jnp.int32) * RPW).astype(jnp.int32)
    bounds = jnp.concatenate([bounds, jnp.zeros(7, jnp.int32)])

    pad_e = NNZ_PAD - NNZ
    colp = jnp.concatenate([pool_col.astype(jnp.int32),
                            jnp.zeros(pad_e, jnp.int32)])
    valp = jnp.concatenate([pool_val, jnp.zeros(pad_e, jnp.float32)])
    rowp = jnp.concatenate([rowi, jnp.zeros(pad_e, jnp.int32)])

    pooled = _pool(h, colp, valp, rowp, bounds)
    return pooled[:N_DOWN][None]


# speedup vs baseline: 2.7628x; 1.0151x over previous
import functools

import jax
import jax.numpy as jnp
from jax import lax
from jax.experimental import pallas as pl
from jax.experimental.pallas import tpu as pltpu
from jax.experimental.pallas import tpu_sc as plsc

N_NODES = 100000
N_DOWN = 25000
SPIRAL_LEN = 9
IN_C = 64
OUT_C = 64
NNZ = 100000

N_SEC = 102400

NW = 32
NPW = 3136
N_PAD = NW * NPW
CB = 112

RPW = 784
OUT_PAD = NW * RPW
CE = 128
NNZ_PAD = NNZ + 2 * CE

_mesh = plsc.VectorSubcoreMesh(core_axis_name="c", subcore_axis_name="s")


def _sget(ref, i):
    return ref[pl.ds(i, 16)][0]


def _mm_body(xt_ref, w_ref, o_ref):
    o_ref[...] = lax.dot_general(
        xt_ref[...], w_ref[0],
        dimension_numbers=(((0,), (0,)), ((), ())),
        preferred_element_type=jnp.float32)


def _project(xt, w3):
    bn = 12800
    nb = N_SEC // bn
    return pl.pallas_call(
        _mm_body,
        grid=(nb, SPIRAL_LEN),
        in_specs=[
            pl.BlockSpec((IN_C, bn), lambda i, l: (0, i)),
            pl.BlockSpec((1, IN_C, 2 * OUT_C), lambda i, l: (l, 0, 0)),
        ],
        out_specs=pl.BlockSpec((bn, 2 * OUT_C), lambda i, l: (l * nb + i, 0)),
        out_shape=jax.ShapeDtypeStruct((N_SEC * SPIRAL_LEN, 2 * OUT_C),
                                       jnp.float32),
    )(xt, w3)


def _fire_gathers(table_hbm, idx_v, h_v, par, sem):
    for l in range(SPIRAL_LEN):
        pltpu.async_copy(
            table_hbm.at[idx_v.at[par, l]],
            h_v.at[par], sem, add=True)


@functools.partial(
    pl.kernel,
    mesh=_mesh,
    compiler_params=pltpu.CompilerParams(use_tc_tiling_on_sc=False),
    out_type=jax.ShapeDtypeStruct((N_PAD, OUT_C), jnp.float32),
    scratch_types=[
        pltpu.VMEM((4, SPIRAL_LEN, CB), jnp.int32),
        pltpu.VMEM((4, CB, OUT_C), jnp.float32),
        pltpu.VMEM((OUT_C,), jnp.float32),
        pltpu.SemaphoreType.DMA,
        pltpu.SemaphoreType.DMA,
        pltpu.SemaphoreType.DMA,
        pltpu.SemaphoreType.DMA,
    ],
)
def _spiral(table_hbm, idxt_hbm, b_hbm, h_hbm, idx_v, h_v, b_v,
            sem0, sem1, sem2, sem3):
    wid = lax.axis_index("s") * 2 + lax.axis_index("c")
    base_node = wid * NPW
    pltpu.sync_copy(b_hbm, b_v)
    nchunks = NPW // CB
    sems = (sem0, sem1, sem2, sem3)

    def zero_buf(par):
        def zb(c, carry):
            for j in range(4):
                h_v[par, c, pl.ds(j * 16, 16)] = jnp.zeros((16,),
                                                           jnp.float32)
            return carry
        lax.fori_loop(0, CB, zb, 0)

    def prefetch(nbase, par):
        zero_buf(par)
        pltpu.sync_copy(idxt_hbm.at[:, pl.ds(nbase, CB)], idx_v.at[par])
        _fire_gathers(table_hbm, idx_v, h_v, par, sems[par])

    for p in range(3):
        prefetch(base_node + p * CB, p)

    def chunk_step(ci, par):
        nbase = base_node + ci * CB

        @pl.when(ci + 3 < nchunks)
        def _pf():
            prefetch(nbase + 3 * CB, (par + 3) % 4)

        for _ in range(SPIRAL_LEN):
            pltpu.make_async_copy(
                table_hbm.at[pl.ds(0, CB)], h_v.at[par], sems[par]).wait()

        def node_body(c, carry2):
            for j in range(4):
                sl = pl.ds(j * 16, 16)
                v = h_v[par, c, sl] + b_v[sl]
                v = jnp.where(v > 0.0, v, jnp.exp(v) - 1.0)
                v_out = v
                h_v[par, c, sl] = v_out
            return carry2

        lax.fori_loop(0, CB, node_body, 0)
        pltpu.sync_copy(h_v.at[par], h_hbm.at[pl.ds(nbase, CB)])

    def quad_body(i, carry):
        for t in range(4):
            chunk_step(4 * i + t, t)
        return carry

    lax.fori_loop(0, nchunks // 4, quad_body, 0)


@functools.partial(
    pl.kernel,
    mesh=_mesh,
    compiler_params=pltpu.CompilerParams(use_tc_tiling_on_sc=False),
    out_type=jax.ShapeDtypeStruct((OUT_PAD, OUT_C), jnp.float32),
    scratch_types=[
        pltpu.VMEM((NW + 1 + 16,), jnp.int32),
        pltpu.VMEM((2, CE + 16), jnp.int32),
        pltpu.VMEM((2, CE + 16), jnp.float32),
        pltpu.VMEM((2, CE + 16), jnp.int32),
        pltpu.VMEM((2, CE, OUT_C), jnp.float32),
        pltpu.VMEM((RPW, OUT_C), jnp.float32),
        pltpu.SemaphoreType.DMA,
        pltpu.SemaphoreType.DMA,
        pltpu.SemaphoreType.DMA,
        pltpu.SemaphoreType.DMA,
    ],
)
def _pool(h_hbm, col_hbm, val_hbm, row_hbm, bnd_hbm, out_hbm,
          bnd_v, col_v, val_v, row_v, g_v, acc_v,
          gsem0, gsem1, msem0, msem1):
    wid = lax.axis_index("s") * 2 + lax.axis_index("c")
    rbase = wid * RPW
    pltpu.sync_copy(bnd_hbm, bnd_v.at[pl.ds(0, 40)])
    k0 = _sget(bnd_v, wid)
    k1 = _sget(bnd_v, wid + 1)
    k0a = jnp.bitwise_and(k0, -8)
    nchunks = lax.shift_right_logical(k1 - k0a + (CE - 1), 7)
    gsems = (gsem0, gsem1)
    msems = (msem0, msem1)

    def fire_meta(kc, par):
        for src, dst in ((col_hbm, col_v), (val_hbm, val_v),
                         (row_hbm, row_v)):
            pltpu.make_async_copy(src.at[pl.ds(kc, CE)],
                                  dst.at[par, pl.ds(0, CE)],
                                  msems[par]).start()

    def drain_meta(par):
        for dst in (col_v, val_v, row_v):
            pltpu.make_async_copy(col_hbm.at[pl.ds(0, CE)],
                                  dst.at[par, pl.ds(0, CE)],
                                  msems[par]).wait()

    def fire_gather(par):
        pltpu.make_async_copy(h_hbm.at[col_v.at[par, pl.ds(0, CE)]],
                              g_v.at[par], gsems[par]).start()

    def drain_gather(par):
        pltpu.make_async_copy(h_hbm.at[pl.ds(0, CE)], g_v.at[par],
                              gsems[par]).wait()

    zero16 = jnp.zeros((16,), jnp.float32)

    def zero_body(r, carry):
        for j in range(4):
            acc_v[r, pl.ds(j * 16, 16)] = zero16
        return carry

    @pl.when(nchunks > 0)
    def _prologue():
        fire_meta(pl.multiple_of(k0a, 8), 0)
        drain_meta(0)
        fire_gather(0)

    lax.fori_loop(0, RPW, zero_body, 0)

    iota16 = lax.iota(jnp.int32, 16)

    def chunk_step(ci, par):
        @pl.when(ci < nchunks)
        def _step():
            kc = pl.multiple_of(k0a + ci * CE, 8)

            @pl.when(ci + 1 < nchunks)
            def _fm():
                fire_meta(pl.multiple_of(kc + CE, 8), 1 - par)

            drain_gather(par)

            def grp_body(g, carry2):
                e0 = g * 16
                kg = kc + e0 + iota16
                ok = jnp.logical_and(kg >= k0, kg < k1)
                vm16 = jnp.where(ok, val_v[par, pl.ds(e0, 16)], 0.0)
                r16 = jnp.clip(row_v[par, pl.ds(e0, 16)] - rbase,
                               0, RPW - 1)
                for j in range(16):
                    rl = r16[j]
                    vm = vm16[j]
                    for q in range(4):
                        sl = pl.ds(q * 16, 16)
                        acc_v[rl, sl] = (acc_v[rl, sl]
                                         + vm * g_v[par, e0 + j, sl])
                return carry2

            lax.fori_loop(0, CE // 16, grp_body, 0)

            @pl.when(ci + 1 < nchunks)
            def _fg():
                drain_meta(1 - par)
                fire_gather(1 - par)

    def pair_body(i, carry):
        chunk_step(2 * i, 0)
        chunk_step(2 * i + 1, 1)
        return carry

    lax.fori_loop(0, lax.shift_right_logical(nchunks + 1, 1), pair_body, 0)
    pltpu.sync_copy(acc_v, out_hbm.at[pl.ds(rbase, RPW)])


def kernel(x, spiral_indices, pool_row, pool_col, pool_val, W, b):
    xt = x[0].T
    w3 = jnp.concatenate(
        [W.reshape(SPIRAL_LEN, IN_C, OUT_C),
         jnp.zeros((SPIRAL_LEN, IN_C, OUT_C), jnp.float32)], axis=2)
    table = _project(xt, w3).reshape(N_SEC * SPIRAL_LEN * 2, OUT_C)

    idxt = (spiral_indices.astype(jnp.int32).T
            + jnp.arange(SPIRAL_LEN, dtype=jnp.int32)[:, None] * N_SEC) * 2
    idxt = jnp.pad(idxt, ((0, 0), (0, N_PAD - N_NODES)))
    h = _spiral(table, idxt, b)

    rowi = pool_row.astype(jnp.int32)
    bounds = jnp.searchsorted(
        rowi, jnp.arange(NW + 1, dtype=jnp.int32) * RPW).astype(jnp.int32)
    bounds = jnp.concatenate([bounds, jnp.zeros(7, jnp.int32)])

    pad_e = NNZ_PAD - NNZ
    colp = jnp.concatenate([pool_col.astype(jnp.int32),
                            jnp.zeros(pad_e, jnp.int32)])
    valp = jnp.concatenate([pool_val, jnp.zeros(pad_e, jnp.float32)])
    rowp = jnp.concatenate([rowi, jnp.zeros(pad_e, jnp.int32)])

    pooled = _pool(h, colp, valp, rowp, bounds)
    return pooled[:N_DOWN][None]
